# baseline (speedup 1.0000x reference)
import functools
import math

import jax
import jax.numpy as jnp
from jax import lax
from jax.experimental import pallas as pl
from jax.experimental.pallas import tpu as pltpu
from jax.experimental.pallas import tpu_sc as plsc

_NEG_T = 0.4
_POS_T = 0.5
_TBL_STRIDE = 128
_NW = 32


def _iou_match(anch_ref, gtb_ref, gtc_ref):
    a0 = anch_ref[0:1, :]
    a1 = anch_ref[1:2, :]
    a2 = anch_ref[2:3, :]
    a3 = anch_ref[3:4, :]
    gtb = gtb_ref[0]
    X1 = gtb[:, 0:1]
    Y1 = gtb[:, 1:2]
    GW = gtb[:, 2:3]
    GH = gtb[:, 3:4]
    C = gtc_ref[0]
    X2 = X1 + GW
    Y2 = Y1 + GH

    ix = jnp.maximum(jnp.minimum(a0 + a2, X2) - jnp.maximum(a0, X1), 0.0)
    iy = jnp.maximum(jnp.minimum(a1 + a3, Y2) - jnp.maximum(a1, Y1), 0.0)
    inter = ix * iy
    union = a2 * a3 + GW * GH - inter
    iou = jnp.where(union > 0.0, inter / jnp.where(union > 0.0, union, 1.0), 0.0)

    mx = jnp.max(iou, axis=0, keepdims=True)
    iota = jax.lax.broadcasted_iota(jnp.int32, iou.shape, 0)
    cand = jnp.where(iou == mx, iota, _TBL_STRIDE)
    fidx = jnp.min(cand, axis=0, keepdims=True)
    return (a0, a1, a2, a3), (X1, Y1, X2, Y2, C), iota, mx, fidx


def _encode_coefs(a, inv_h, inv_w):
    a0, a1, a2, a3 = a
    cx0 = (a0 + a2) * 0.5
    cy0 = (a1 + a3) * 0.5
    r0 = 1.0 / (a2 - a0)
    r1 = 1.0 / (a3 - a1)
    arows = [cx0 * r0, cy0 * r1, -cx0 * r0, -cy0 * r1]
    brows = [-r0 * inv_h, -r1 * inv_w, r0 * inv_h, r1 * inv_w]
    return arows, brows


def _match_kernel(anch_ref, gtb_ref, gtc_ref, gtr_ref, idx_ref, tbl_ref,
                  coef_ref, *, inv_h, inv_w):
    a, _, _, mx, fidx = _iou_match(anch_ref, gtb_ref, gtc_ref)

    variant = ((mx < _POS_T).astype(jnp.int32)
               + (mx < _NEG_T).astype(jnp.int32))
    idx_ref[0] = fidx + variant * _TBL_STRIDE

    xr = gtr_ref[0, 0:1, :]
    yr = gtr_ref[0, 1:2, :]
    wr = gtr_ref[0, 2:3, :]
    hr = gtr_ref[0, 3:4, :]
    cr = gtr_ref[0, 4:5, :]
    box4 = jnp.concatenate([yr, xr, yr + hr, xr + wr], axis=0)
    for v in range(3):
        tbl_ref[0, 0:4, v * _TBL_STRIDE:(v + 1) * _TBL_STRIDE] = box4
    tbl_ref[0, 4:5, 0:_TBL_STRIDE] = cr
    tbl_ref[0, 4:5, _TBL_STRIDE:2 * _TBL_STRIDE] = jnp.full(
        (1, _TBL_STRIDE), -2.0, jnp.float32)
    tbl_ref[0, 4:5, 2 * _TBL_STRIDE:3 * _TBL_STRIDE] = jnp.full(
        (1, _TBL_STRIDE), -1.0, jnp.float32)

    arows, brows = _encode_coefs(a, inv_h, inv_w)
    zrow = jnp.zeros((4, arows[0].shape[1]), jnp.float32)
    coef_ref[0] = jnp.concatenate(arows + [zrow], axis=0)
    coef_ref[1] = jnp.concatenate(brows + [zrow], axis=0)


def _fused_kernel(anch_ref, gtb_ref, gtc_ref, box_ref, cls_ref,
                  *, inv_h, inv_w):
    a, g, iota, mx, fidx = _iou_match(anch_ref, gtb_ref, gtc_ref)
    X1, Y1, X2, Y2, C = g

    onehot = (iota == fidx).astype(jnp.float32)
    cols = jnp.concatenate([Y1, X1, Y2, X2, C], axis=1)
    m5 = jax.lax.dot_general(cols, onehot, (((0,), (0,)), ((), ())),
                             preferred_element_type=jnp.float32)
    gy = m5[0:1]
    gx = m5[1:2]
    gyh = m5[2:3]
    gxw = m5[3:4]
    gc = m5[4:5]

    (a0, a1, a2, a3) = a
    arows, brows = _encode_coefs(a, inv_h, inv_w)
    p10 = arows[0] + brows[0] * gy
    p11 = arows[1] + brows[1] * gx
    p20 = arows[2] + brows[2] * gyh
    p21 = arows[3] + brows[3] * gxw

    cls = jnp.where(mx >= _POS_T, gc,
                    jnp.where(mx >= _NEG_T, -2.0, -1.0))

    nan = (jnp.isnan(p10) | jnp.isnan(p11) | jnp.isnan(p20)
           | jnp.isnan(p21) | jnp.isnan(cls))
    p10 = jnp.where(nan, -2.0, p10)
    p11 = jnp.where(nan, -2.0, p11)
    p20 = jnp.where(nan, -2.0, p20)
    p21 = jnp.where(nan, -2.0, p21)
    cls = jnp.where(nan, -2.0, cls)

    box_ref[0] = jnp.concatenate([p10, p11, p20, p21], axis=0)
    cls_ref[0] = cls


def _make_sc_assign(nr, wpb):
    nact = 4 * wpb

    def _sc_assign(tbl_hbm, idx_hbm, coef_hbm, box_hbm, cls_hbm,
                   idx_v, tbl_v, coef_v, obox_v, ocls_v, sem, sem2, sem3):
        wid = lax.axis_index("s") * 2 + lax.axis_index("c")

        @pl.when(wid < nact)
        def _():
            bw = lax.div(wid, wpb)
            aoff = pl.multiple_of(lax.rem(wid, wpb) * nr, 128)

            c1 = pltpu.async_copy(idx_hbm.at[bw, 0, pl.ds(aoff, nr)], idx_v,
                                  sem)
            c2 = pltpu.async_copy(tbl_hbm.at[bw], tbl_v, sem2)
            c3 = pltpu.async_copy(coef_hbm.at[:, :, pl.ds(aoff, nr)],
                                  coef_v, sem3)
            c1.wait()
            c2.wait()
            c3.wait()

            @pl.loop(0, nr // 16)
            def _(j):
                row0 = j * 16
                idx16 = idx_v[pl.ds(row0, 16)]
                for c in range(4):
                    cidx = jnp.full((16,), c, jnp.int32)
                    g = plsc.load_gather(tbl_v, [cidx, idx16])
                    a = coef_v[0, c, pl.ds(row0, 16)]
                    bb = coef_v[1, c, pl.ds(row0, 16)]
                    obox_v[c, pl.ds(row0, 16)] = a + bb * g
                ocls_v[0, pl.ds(row0, 16)] = plsc.load_gather(
                    tbl_v, [jnp.full((16,), 4, jnp.int32), idx16])

            pltpu.sync_copy(obox_v, box_hbm.at[bw, :, pl.ds(aoff, nr)])
            pltpu.sync_copy(ocls_v, cls_hbm.at[bw, :, pl.ds(aoff, nr)])

    return _sc_assign


def kernel(images, gt_boxes, gt_classes, anchor_boxes):
    B, N = gt_boxes.shape[0], gt_boxes.shape[1]
    M = anchor_boxes.shape[0]
    H, W = images.shape[1], images.shape[2]

    bsc = B // 2
    btc = B - bsc
    nr = 768
    wpb = M // nr

    anch_t = anchor_boxes.T
    inv = dict(inv_h=1.0 / H, inv_w=1.0 / W)

    gt5 = jnp.concatenate([gt_boxes[:bsc], gt_classes[:bsc]], axis=-1)
    gt_rows = jnp.pad(jnp.transpose(gt5, (0, 2, 1)),
                      ((0, 0), (0, 3), (0, _TBL_STRIDE - N)))

    gidx, tbl, coef = pl.pallas_call(
        functools.partial(_match_kernel, **inv),
        grid=(bsc,),
        in_specs=[
            pl.BlockSpec((4, M), lambda b: (0, 0)),
            pl.BlockSpec((1, N, 4), lambda b: (b, 0, 0)),
            pl.BlockSpec((1, N, 1), lambda b: (b, 0, 0)),
            pl.BlockSpec((1, 8, _TBL_STRIDE), lambda b: (b, 0, 0)),
        ],
        out_specs=[
            pl.BlockSpec((1, 1, M), lambda b: (b, 0, 0)),
            pl.BlockSpec((1, 8, 3 * _TBL_STRIDE), lambda b: (b, 0, 0)),
            pl.BlockSpec((2, 8, M), lambda b: (0, 0, 0)),
        ],
        out_shape=[
            jax.ShapeDtypeStruct((bsc, 1, M), jnp.int32),
            jax.ShapeDtypeStruct((bsc, 8, 3 * _TBL_STRIDE), jnp.float32),
            jax.ShapeDtypeStruct((2, 8, M), jnp.float32),
        ],
    )(anch_t, gt_boxes[:bsc], gt_classes[:bsc], gt_rows)

    mesh = plsc.VectorSubcoreMesh(core_axis_name="c", subcore_axis_name="s")
    sc = functools.partial(
        pl.kernel, mesh=mesh,
        compiler_params=pltpu.CompilerParams(needs_layout_passes=False,
                                             use_tc_tiling_on_sc=True),
        out_type=[
            jax.ShapeDtypeStruct((bsc, 4, M), jnp.float32),
            jax.ShapeDtypeStruct((bsc, 1, M), jnp.float32),
        ],
        scratch_types=[
            pltpu.VMEM((nr,), jnp.int32),
            pltpu.VMEM((8, 3 * _TBL_STRIDE), jnp.float32),
            pltpu.VMEM((2, 8, nr), jnp.float32),
            pltpu.VMEM((4, nr), jnp.float32),
            pltpu.VMEM((1, nr), jnp.float32),
            pltpu.SemaphoreType.DMA,
            pltpu.SemaphoreType.DMA,
            pltpu.SemaphoreType.DMA,
        ],
    )(_make_sc_assign(nr, wpb))
    box0, cls0 = sc(tbl, gidx, coef)

    boxt, clst = pl.pallas_call(
        functools.partial(_fused_kernel, **inv),
        grid=(btc,),
        in_specs=[
            pl.BlockSpec((4, M), lambda b: (0, 0)),
            pl.BlockSpec((1, N, 4), lambda b: (b, 0, 0)),
            pl.BlockSpec((1, N, 1), lambda b: (b, 0, 0)),
        ],
        out_specs=[
            pl.BlockSpec((1, 4, M), lambda b: (b, 0, 0)),
            pl.BlockSpec((1, 1, M), lambda b: (b, 0, 0)),
        ],
        out_shape=[
            jax.ShapeDtypeStruct((btc, 4, M), jnp.float32),
            jax.ShapeDtypeStruct((btc, 1, M), jnp.float32),
        ],
    )(anch_t, gt_boxes[bsc:], gt_classes[bsc:])

    box = jnp.concatenate([box0, boxt], axis=0).transpose(0, 2, 1)
    cls = jnp.concatenate([cls0, clst], axis=0)[:, 0, :]
    return box, cls


# speedup vs baseline: 1.0125x; 1.0125x over previous
import functools
import math

import jax
import jax.numpy as jnp
from jax import lax
from jax.experimental import pallas as pl
from jax.experimental.pallas import tpu as pltpu
from jax.experimental.pallas import tpu_sc as plsc

_NEG_T = 0.4
_POS_T = 0.5
_TBL_STRIDE = 128
_NW = 32


def _iou_match(anch_ref, gtb_ref, gtc_ref):
    a0 = anch_ref[0:1, :]
    a1 = anch_ref[1:2, :]
    a2 = anch_ref[2:3, :]
    a3 = anch_ref[3:4, :]
    gtb = gtb_ref[0]
    X1 = gtb[:, 0:1]
    Y1 = gtb[:, 1:2]
    GW = gtb[:, 2:3]
    GH = gtb[:, 3:4]
    C = gtc_ref[0]
    X2 = X1 + GW
    Y2 = Y1 + GH

    ix = jnp.maximum(jnp.minimum(a0 + a2, X2) - jnp.maximum(a0, X1), 0.0)
    iy = jnp.maximum(jnp.minimum(a1 + a3, Y2) - jnp.maximum(a1, Y1), 0.0)
    inter = ix * iy
    union = a2 * a3 + GW * GH - inter
    iou = jnp.where(union > 0.0, inter / jnp.where(union > 0.0, union, 1.0), 0.0)

    mx = jnp.max(iou, axis=0, keepdims=True)
    iota = jax.lax.broadcasted_iota(jnp.int32, iou.shape, 0)
    cand = jnp.where(iou == mx, iota, _TBL_STRIDE)
    fidx = jnp.min(cand, axis=0, keepdims=True)
    return (a0, a1, a2, a3), (X1, Y1, X2, Y2, C), iota, mx, fidx


def _encode_coefs(a, inv_h, inv_w):
    a0, a1, a2, a3 = a
    cx0 = (a0 + a2) * 0.5
    cy0 = (a1 + a3) * 0.5
    r0 = 1.0 / (a2 - a0)
    r1 = 1.0 / (a3 - a1)
    arows = [cx0 * r0, cy0 * r1, -cx0 * r0, -cy0 * r1]
    brows = [-r0 * inv_h, -r1 * inv_w, r0 * inv_h, r1 * inv_w]
    return arows, brows


def _match_kernel(anch_ref, gtb_ref, gtc_ref, idx_ref, tbl_ref, coef_ref,
                  *, inv_h, inv_w):
    n = gtb_ref.shape[1]
    a, g, _, mx, fidx = _iou_match(anch_ref, gtb_ref, gtc_ref)
    X1, Y1, X2, Y2, C = g

    variant = ((mx < _POS_T).astype(jnp.int32)
               + (mx < _NEG_T).astype(jnp.int32))
    idx_ref[0] = fidx + variant * _TBL_STRIDE

    zcol = jnp.zeros((n, 11), jnp.float32)
    base = jnp.concatenate([Y1, X1, Y2, X2], axis=1)
    tbl_ref[0, 0:n] = jnp.concatenate([base, C, zcol], axis=1)
    tbl_ref[0, _TBL_STRIDE:_TBL_STRIDE + n] = jnp.concatenate(
        [base, jnp.full((n, 1), -2.0, jnp.float32), zcol], axis=1)
    tbl_ref[0, 2 * _TBL_STRIDE:2 * _TBL_STRIDE + n] = jnp.concatenate(
        [base, jnp.full((n, 1), -1.0, jnp.float32), zcol], axis=1)

    arows, brows = _encode_coefs(a, inv_h, inv_w)
    zrow = jnp.zeros((4, arows[0].shape[1]), jnp.float32)
    coef_ref[0] = jnp.concatenate(arows + [zrow], axis=0)
    coef_ref[1] = jnp.concatenate(brows + [zrow], axis=0)


def _fused_kernel(anch_ref, gtb_ref, gtc_ref, box_ref, cls_ref,
                  *, inv_h, inv_w):
    a, g, iota, mx, fidx = _iou_match(anch_ref, gtb_ref, gtc_ref)
    X1, Y1, X2, Y2, C = g

    onehot = (iota == fidx).astype(jnp.float32)
    cols = jnp.concatenate([Y1, X1, Y2, X2, C], axis=1)
    m5 = jax.lax.dot_general(cols, onehot, (((0,), (0,)), ((), ())),
                             preferred_element_type=jnp.float32)
    gy = m5[0:1]
    gx = m5[1:2]
    gyh = m5[2:3]
    gxw = m5[3:4]
    gc = m5[4:5]

    (a0, a1, a2, a3) = a
    arows, brows = _encode_coefs(a, inv_h, inv_w)
    p10 = arows[0] + brows[0] * gy
    p11 = arows[1] + brows[1] * gx
    p20 = arows[2] + brows[2] * gyh
    p21 = arows[3] + brows[3] * gxw

    cls = jnp.where(mx >= _POS_T, gc,
                    jnp.where(mx >= _NEG_T, -2.0, -1.0))

    nan = (jnp.isnan(p10) | jnp.isnan(p11) | jnp.isnan(p20)
           | jnp.isnan(p21) | jnp.isnan(cls))
    p10 = jnp.where(nan, -2.0, p10)
    p11 = jnp.where(nan, -2.0, p11)
    p20 = jnp.where(nan, -2.0, p20)
    p21 = jnp.where(nan, -2.0, p21)
    cls = jnp.where(nan, -2.0, cls)

    box_ref[0] = jnp.concatenate([p10, p11, p20, p21], axis=0)
    cls_ref[0] = cls


def _make_sc_assign(nr, wpb, nact):

    def _sc_assign(tbl_hbm, idx_hbm, coef_hbm, box_hbm, cls_hbm,
                   idx_v, tbl_v, coef_v, obox_v, ocls_v, sem, sem2, sem3):
        wid = lax.axis_index("s") * 2 + lax.axis_index("c")

        @pl.when(wid < nact)
        def _():
            bw = lax.div(wid, wpb)
            aoff = pl.multiple_of(lax.rem(wid, wpb) * nr, 128)

            c1 = pltpu.async_copy(idx_hbm.at[bw, 0, pl.ds(aoff, nr)], idx_v,
                                  sem)
            c2 = pltpu.async_copy(tbl_hbm.at[bw], tbl_v, sem2)
            c3 = pltpu.async_copy(coef_hbm.at[:, :, pl.ds(aoff, nr)],
                                  coef_v, sem3)
            c1.wait()
            c2.wait()
            c3.wait()

            @pl.loop(0, nr // 16)
            def _(j):
                row0 = j * 16
                idx16 = idx_v[pl.ds(row0, 16)]
                for c in range(4):
                    cidx = jnp.full((16,), c, jnp.int32)
                    g = plsc.load_gather(tbl_v, [idx16, cidx])
                    a = coef_v[0, c, pl.ds(row0, 16)]
                    bb = coef_v[1, c, pl.ds(row0, 16)]
                    obox_v[c, pl.ds(row0, 16)] = a + bb * g
                ocls_v[0, pl.ds(row0, 16)] = plsc.load_gather(
                    tbl_v, [idx16, jnp.full((16,), 4, jnp.int32)])

            pltpu.sync_copy(obox_v, box_hbm.at[bw, :, pl.ds(aoff, nr)])
            pltpu.sync_copy(ocls_v, cls_hbm.at[bw, :, pl.ds(aoff, nr)])

    return _sc_assign


def kernel(images, gt_boxes, gt_classes, anchor_boxes):
    B, N = gt_boxes.shape[0], gt_boxes.shape[1]
    M = anchor_boxes.shape[0]
    H, W = images.shape[1], images.shape[2]

    bsc = B // 4
    btc = B - bsc
    nr = 768
    wpb = M // nr

    anch_t = anchor_boxes.T
    inv = dict(inv_h=1.0 / H, inv_w=1.0 / W)

    gidx, tbl, coef = pl.pallas_call(
        functools.partial(_match_kernel, **inv),
        grid=(bsc,),
        in_specs=[
            pl.BlockSpec((4, M), lambda b: (0, 0)),
            pl.BlockSpec((1, N, 4), lambda b: (b, 0, 0)),
            pl.BlockSpec((1, N, 1), lambda b: (b, 0, 0)),
        ],
        out_specs=[
            pl.BlockSpec((1, 1, M), lambda b: (b, 0, 0)),
            pl.BlockSpec((1, 3 * _TBL_STRIDE, 16), lambda b: (b, 0, 0)),
            pl.BlockSpec((2, 8, M), lambda b: (0, 0, 0)),
        ],
        out_shape=[
            jax.ShapeDtypeStruct((bsc, 1, M), jnp.int32),
            jax.ShapeDtypeStruct((bsc, 3 * _TBL_STRIDE, 16), jnp.float32),
            jax.ShapeDtypeStruct((2, 8, M), jnp.float32),
        ],
    )(anch_t, gt_boxes[:bsc], gt_classes[:bsc])

    mesh = plsc.VectorSubcoreMesh(core_axis_name="c", subcore_axis_name="s")
    sc = functools.partial(
        pl.kernel, mesh=mesh,
        compiler_params=pltpu.CompilerParams(needs_layout_passes=False,
                                             use_tc_tiling_on_sc=True),
        out_type=[
            jax.ShapeDtypeStruct((bsc, 4, M), jnp.float32),
            jax.ShapeDtypeStruct((bsc, 1, M), jnp.float32),
        ],
        scratch_types=[
            pltpu.VMEM((nr,), jnp.int32),
            pltpu.VMEM((3 * _TBL_STRIDE, 16), jnp.float32),
            pltpu.VMEM((2, 8, nr), jnp.float32),
            pltpu.VMEM((4, nr), jnp.float32),
            pltpu.VMEM((1, nr), jnp.float32),
            pltpu.SemaphoreType.DMA,
            pltpu.SemaphoreType.DMA,
            pltpu.SemaphoreType.DMA,
        ],
    )(_make_sc_assign(nr, wpb, bsc * wpb))
    box0, cls0 = sc(tbl, gidx, coef)

    boxt, clst = pl.pallas_call(
        functools.partial(_fused_kernel, **inv),
        grid=(btc,),
        in_specs=[
            pl.BlockSpec((4, M), lambda b: (0, 0)),
            pl.BlockSpec((1, N, 4), lambda b: (b, 0, 0)),
            pl.BlockSpec((1, N, 1), lambda b: (b, 0, 0)),
        ],
        out_specs=[
            pl.BlockSpec((1, 4, M), lambda b: (b, 0, 0)),
            pl.BlockSpec((1, 1, M), lambda b: (b, 0, 0)),
        ],
        out_shape=[
            jax.ShapeDtypeStruct((btc, 4, M), jnp.float32),
            jax.ShapeDtypeStruct((btc, 1, M), jnp.float32),
        ],
    )(anch_t, gt_boxes[bsc:], gt_classes[bsc:])

    box = jnp.concatenate([box0, boxt], axis=0).transpose(0, 2, 1)
    cls = jnp.concatenate([cls0, clst], axis=0)[:, 0, :]
    return box, cls


# speedup vs baseline: 1.0142x; 1.0017x over previous
import functools
import math

import jax
import jax.numpy as jnp
from jax import lax
from jax.experimental import pallas as pl
from jax.experimental.pallas import tpu as pltpu
from jax.experimental.pallas import tpu_sc as plsc

_NEG_T = 0.4
_POS_T = 0.5
_TBL_STRIDE = 128
_NW = 32


def _iou_match(anch_ref, gtb_ref, gtc_ref):
    a0 = anch_ref[0:1, :]
    a1 = anch_ref[1:2, :]
    a2 = anch_ref[2:3, :]
    a3 = anch_ref[3:4, :]
    gtb = gtb_ref[0]
    X1 = gtb[:, 0:1]
    Y1 = gtb[:, 1:2]
    GW = gtb[:, 2:3]
    GH = gtb[:, 3:4]
    C = gtc_ref[0]
    X2 = X1 + GW
    Y2 = Y1 + GH

    ix = jnp.maximum(jnp.minimum(a0 + a2, X2) - jnp.maximum(a0, X1), 0.0)
    iy = jnp.maximum(jnp.minimum(a1 + a3, Y2) - jnp.maximum(a1, Y1), 0.0)
    inter = ix * iy
    union = a2 * a3 + GW * GH - inter
    iou = jnp.where(union > 0.0, inter / jnp.where(union > 0.0, union, 1.0), 0.0)

    mx = jnp.max(iou, axis=0, keepdims=True)
    iota = jax.lax.broadcasted_iota(jnp.int32, iou.shape, 0)
    cand = jnp.where(iou == mx, iota, _TBL_STRIDE)
    fidx = jnp.min(cand, axis=0, keepdims=True)
    return (a0, a1, a2, a3), (X1, Y1, X2, Y2, C), iota, mx, fidx


def _encode_coefs(a, inv_h, inv_w):
    a0, a1, a2, a3 = a
    cx0 = (a0 + a2) * 0.5
    cy0 = (a1 + a3) * 0.5
    r0 = 1.0 / (a2 - a0)
    r1 = 1.0 / (a3 - a1)
    arows = [cx0 * r0, cy0 * r1, -cx0 * r0, -cy0 * r1]
    brows = [-r0 * inv_h, -r1 * inv_w, r0 * inv_h, r1 * inv_w]
    return arows, brows


def _match_kernel(anch_ref, gtb_ref, gtc_ref, idx_ref, tbl_ref, coef_ref,
                  *, inv_h, inv_w):
    n = gtb_ref.shape[1]
    a, g, _, mx, fidx = _iou_match(anch_ref, gtb_ref, gtc_ref)
    X1, Y1, X2, Y2, C = g

    variant = ((mx < _POS_T).astype(jnp.int32)
               + (mx < _NEG_T).astype(jnp.int32))
    idx_ref[0] = fidx + variant * _TBL_STRIDE

    zcol = jnp.zeros((n, 11), jnp.float32)
    base = jnp.concatenate([Y1, X1, Y2, X2], axis=1)
    tbl_ref[0, 0:n] = jnp.concatenate([base, C, zcol], axis=1)
    tbl_ref[0, _TBL_STRIDE:_TBL_STRIDE + n] = jnp.concatenate(
        [base, jnp.full((n, 1), -2.0, jnp.float32), zcol], axis=1)
    tbl_ref[0, 2 * _TBL_STRIDE:2 * _TBL_STRIDE + n] = jnp.concatenate(
        [base, jnp.full((n, 1), -1.0, jnp.float32), zcol], axis=1)

    arows, brows = _encode_coefs(a, inv_h, inv_w)
    zrow = jnp.zeros((4, arows[0].shape[1]), jnp.float32)
    coef_ref[0] = jnp.concatenate(arows + [zrow], axis=0)
    coef_ref[1] = jnp.concatenate(brows + [zrow], axis=0)


def _fused_kernel(anch_ref, gtb_ref, gtc_ref, box_ref, cls_ref,
                  *, inv_h, inv_w):
    a, g, iota, mx, fidx = _iou_match(anch_ref, gtb_ref, gtc_ref)
    X1, Y1, X2, Y2, C = g

    onehot = (iota == fidx).astype(jnp.float32)
    cols = jnp.concatenate([Y1, X1, Y2, X2, C], axis=1)
    m5 = jax.lax.dot_general(cols, onehot, (((0,), (0,)), ((), ())),
                             preferred_element_type=jnp.float32)
    gy = m5[0:1]
    gx = m5[1:2]
    gyh = m5[2:3]
    gxw = m5[3:4]
    gc = m5[4:5]

    (a0, a1, a2, a3) = a
    arows, brows = _encode_coefs(a, inv_h, inv_w)
    p10 = arows[0] + brows[0] * gy
    p11 = arows[1] + brows[1] * gx
    p20 = arows[2] + brows[2] * gyh
    p21 = arows[3] + brows[3] * gxw

    cls = jnp.where(mx >= _POS_T, gc,
                    jnp.where(mx >= _NEG_T, -2.0, -1.0))

    nan = (jnp.isnan(p10) | jnp.isnan(p11) | jnp.isnan(p20)
           | jnp.isnan(p21) | jnp.isnan(cls))
    p10 = jnp.where(nan, -2.0, p10)
    p11 = jnp.where(nan, -2.0, p11)
    p20 = jnp.where(nan, -2.0, p20)
    p21 = jnp.where(nan, -2.0, p21)
    cls = jnp.where(nan, -2.0, cls)

    box_ref[0] = jnp.concatenate([p10, p11, p20, p21], axis=0)
    cls_ref[0] = cls


def _make_sc_assign(nr, wpb, nact):

    def _sc_assign(tbl_hbm, idx_hbm, coef_hbm, box_hbm, cls_hbm,
                   idx_v, tbl_v, coef_v, obox_v, ocls_v, sem, sem2, sem3):
        wid = lax.axis_index("s") * 2 + lax.axis_index("c")

        @pl.when(wid < nact)
        def _():
            bw = lax.div(wid, wpb)
            aoff = pl.multiple_of(lax.rem(wid, wpb) * nr, 128)

            c1 = pltpu.async_copy(idx_hbm.at[bw, 0, pl.ds(aoff, nr)], idx_v,
                                  sem)
            c2 = pltpu.async_copy(tbl_hbm.at[bw], tbl_v, sem2)
            c3 = pltpu.async_copy(coef_hbm.at[:, :, pl.ds(aoff, nr)],
                                  coef_v, sem3)
            c1.wait()
            c2.wait()
            c3.wait()

            @pl.loop(0, nr // 16)
            def _(j):
                row0 = j * 16
                idx16 = idx_v[pl.ds(row0, 16)]
                for c in range(4):
                    cidx = jnp.full((16,), c, jnp.int32)
                    g = plsc.load_gather(tbl_v, [idx16, cidx])
                    a = coef_v[0, c, pl.ds(row0, 16)]
                    bb = coef_v[1, c, pl.ds(row0, 16)]
                    obox_v[c, pl.ds(row0, 16)] = a + bb * g
                ocls_v[0, pl.ds(row0, 16)] = plsc.load_gather(
                    tbl_v, [idx16, jnp.full((16,), 4, jnp.int32)])

            pltpu.sync_copy(obox_v, box_hbm.at[bw, :, pl.ds(aoff, nr)])
            pltpu.sync_copy(ocls_v, cls_hbm.at[bw, :, pl.ds(aoff, nr)])

    return _sc_assign


def kernel(images, gt_boxes, gt_classes, anchor_boxes):
    B, N = gt_boxes.shape[0], gt_boxes.shape[1]
    M = anchor_boxes.shape[0]
    H, W = images.shape[1], images.shape[2]

    bsc = B // 2
    btc = B - bsc
    nr = 768
    wpb = M // nr

    anch_t = anchor_boxes.T
    inv = dict(inv_h=1.0 / H, inv_w=1.0 / W)

    gidx, tbl, coef = pl.pallas_call(
        functools.partial(_match_kernel, **inv),
        grid=(bsc,),
        in_specs=[
            pl.BlockSpec((4, M), lambda b: (0, 0)),
            pl.BlockSpec((1, N, 4), lambda b: (b, 0, 0)),
            pl.BlockSpec((1, N, 1), lambda b: (b, 0, 0)),
        ],
        out_specs=[
            pl.BlockSpec((1, 1, M), lambda b: (b, 0, 0)),
            pl.BlockSpec((1, 3 * _TBL_STRIDE, 16), lambda b: (b, 0, 0)),
            pl.BlockSpec((2, 8, M), lambda b: (0, 0, 0)),
        ],
        out_shape=[
            jax.ShapeDtypeStruct((bsc, 1, M), jnp.int32),
            jax.ShapeDtypeStruct((bsc, 3 * _TBL_STRIDE, 16), jnp.float32),
            jax.ShapeDtypeStruct((2, 8, M), jnp.float32),
        ],
    )(anch_t, gt_boxes[:bsc], gt_classes[:bsc])

    mesh = plsc.VectorSubcoreMesh(core_axis_name="c", subcore_axis_name="s")
    sc = functools.partial(
        pl.kernel, mesh=mesh,
        compiler_params=pltpu.CompilerParams(needs_layout_passes=False,
                                             use_tc_tiling_on_sc=True),
        out_type=[
            jax.ShapeDtypeStruct((bsc, 4, M), jnp.float32),
            jax.ShapeDtypeStruct((bsc, 1, M), jnp.float32),
        ],
        scratch_types=[
            pltpu.VMEM((nr,), jnp.int32),
            pltpu.VMEM((3 * _TBL_STRIDE, 16), jnp.float32),
            pltpu.VMEM((2, 8, nr), jnp.float32),
            pltpu.VMEM((4, nr), jnp.float32),
            pltpu.VMEM((1, nr), jnp.float32),
            pltpu.SemaphoreType.DMA,
            pltpu.SemaphoreType.DMA,
            pltpu.SemaphoreType.DMA,
        ],
    )(_make_sc_assign(nr, wpb, bsc * wpb))
    box0, cls0 = sc(tbl, gidx, coef)

    boxt, clst = pl.pallas_call(
        functools.partial(_fused_kernel, **inv),
        grid=(btc,),
        in_specs=[
            pl.BlockSpec((4, M), lambda b: (0, 0)),
            pl.BlockSpec((1, N, 4), lambda b: (b, 0, 0)),
            pl.BlockSpec((1, N, 1), lambda b: (b, 0, 0)),
        ],
        out_specs=[
            pl.BlockSpec((1, 4, M), lambda b: (b, 0, 0)),
            pl.BlockSpec((1, 1, M), lambda b: (b, 0, 0)),
        ],
        out_shape=[
            jax.ShapeDtypeStruct((btc, 4, M), jnp.float32),
            jax.ShapeDtypeStruct((btc, 1, M), jnp.float32),
        ],
    )(anch_t, gt_boxes[bsc:], gt_classes[bsc:])

    box = jnp.concatenate([box0, boxt], axis=0).transpose(0, 2, 1)
    cls = jnp.concatenate([cls0, clst], axis=0)[:, 0, :]
    return box, cls
